# baseline (speedup 1.0000x reference)
import functools

import jax
import jax.numpy as jnp
from jax import lax
from jax.experimental import pallas as pl
from jax.experimental.pallas import tpu as pltpu
from jax.experimental.pallas import tpu_sc as plsc

B = 4096
D = 16
VOCAB_ROWS = 125000


def _build_gather():
    info = plsc.get_sparse_core_info()
    nc, ns = info.num_cores, info.num_subcores
    nw = nc * ns
    bpw = B // nw
    mesh = plsc.VectorSubcoreMesh(core_axis_name="c", subcore_axis_name="s")

    @functools.partial(
        pl.kernel,
        mesh=mesh,
        out_type=[
            jax.ShapeDtypeStruct((B, 128), jnp.float32),
            jax.ShapeDtypeStruct((B, 128), jnp.float32),
        ],
        scratch_types=[
            pltpu.VMEM((bpw,), jnp.int32),
            pltpu.VMEM((bpw,), jnp.int32),
            pltpu.VMEM((bpw,), jnp.int32),
            pltpu.VMEM((bpw,), jnp.int32),
            pltpu.VMEM((bpw, 128), jnp.float32),
            pltpu.VMEM((bpw, 128), jnp.float32),
            pltpu.SemaphoreType.DMA,
            pltpu.SemaphoreType.DMA,
        ],
    )
    def gather2(tgt_hbm, ctx_hbm, table_hbm, t_out, c_out,
                tidx_v, cidx_v, trow_v, crow_v, trows_v, crows_v,
                sem_t, sem_c):
        wid = lax.axis_index("s") * nc + lax.axis_index("c")
        base = wid * bpw
        pltpu.sync_copy(tgt_hbm.at[pl.ds(base, bpw)], tidx_v)
        pltpu.sync_copy(ctx_hbm.at[pl.ds(base, bpw)], cidx_v)
        for k in range(bpw // 16):
            sl = pl.ds(k * 16, 16)
            trow_v[sl] = lax.shift_right_logical(tidx_v[sl], 3)
            crow_v[sl] = lax.shift_right_logical(cidx_v[sl], 3)
        cp_t = pltpu.async_copy(table_hbm.at[trow_v], trows_v, sem_t)
        cp_c = pltpu.async_copy(table_hbm.at[crow_v], crows_v, sem_c)
        cp_t.wait()
        cp_c.wait()
        pltpu.sync_copy(trows_v, t_out.at[pl.ds(base, bpw)])
        pltpu.sync_copy(crows_v, c_out.at[pl.ds(base, bpw)])

    return gather2


_gather2 = _build_gather()

_BM = 512


def _select16(pad, sel):
    acc = jnp.zeros((pad.shape[0], D), jnp.float32)
    for j in range(8):
        m = (sel == j).astype(jnp.float32)
        acc = acc + pad[:, j * D:(j + 1) * D] * m
    return acc


def _mm_body(t_ref, ts_ref, c_ref, cs_ref, o_ref):
    t_sel = _select16(t_ref[...], ts_ref[...])
    c_sel = _select16(c_ref[...], cs_ref[...])
    o_ref[...] = lax.dot_general(
        t_sel, c_sel,
        dimension_numbers=(((1,), (1,)), ((), ())),
        preferred_element_type=jnp.float32,
    )


_matmul = pl.pallas_call(
    _mm_body,
    grid=(B // _BM,),
    in_specs=[
        pl.BlockSpec((_BM, 128), lambda i: (i, 0)),
        pl.BlockSpec((_BM, 1), lambda i: (i, 0)),
        pl.BlockSpec((B, 128), lambda i: (0, 0)),
        pl.BlockSpec((B, 1), lambda i: (0, 0)),
    ],
    out_specs=pl.BlockSpec((_BM, B), lambda i: (i, 0)),
    out_shape=jax.ShapeDtypeStruct((B, B), jnp.float32),
)


def kernel(target, context, table):
    target = target.astype(jnp.int32)
    context = context.astype(jnp.int32)
    table128 = table.reshape(VOCAB_ROWS, 128)
    t_pad, c_pad = _gather2(target, context, table128)
    sel_t = (target & 7)[:, None]
    sel_c = (context & 7)[:, None]
    return _matmul(t_pad, sel_t, c_pad, sel_c)


# speedup vs baseline: 4.0017x; 4.0017x over previous
import functools

import jax
import jax.numpy as jnp
from jax import lax
from jax.experimental import pallas as pl
from jax.experimental.pallas import tpu as pltpu
from jax.experimental.pallas import tpu_sc as plsc

B = 4096
D = 16
VOCAB = 1000000


def _build_gather():
    info = plsc.get_sparse_core_info()
    nc, ns = info.num_cores, info.num_subcores
    nw = nc * ns
    bpw = B // nw
    nb = 16
    mesh = plsc.VectorSubcoreMesh(core_axis_name="c", subcore_axis_name="s")

    @functools.partial(
        pl.kernel,
        mesh=mesh,
        out_type=[
            jax.ShapeDtypeStruct((B, D, 128), jnp.float32),
            jax.ShapeDtypeStruct((B, D, 128), jnp.float32),
        ],
        scratch_types=[
            pltpu.VMEM((bpw,), jnp.int32),
            pltpu.VMEM((bpw,), jnp.int32),
            pltpu.VMEM((nb, D, 128), jnp.float32),
            pltpu.VMEM((nb, D, 128), jnp.float32),
            pltpu.SemaphoreType.DMA,
            pltpu.SemaphoreType.DMA,
        ],
    )
    def gather2(tgt_hbm, ctx_hbm, tableT_hbm, t_out, c_out,
                tidx_v, cidx_v, tstage, cstage, sem_t, sem_c):
        wid = lax.axis_index("s") * nc + lax.axis_index("c")
        base = wid * bpw
        pltpu.sync_copy(tgt_hbm.at[pl.ds(base, bpw)], tidx_v)
        pltpu.sync_copy(ctx_hbm.at[pl.ds(base, bpw)], cidx_v)
        for c in range(bpw // nb):
            tv = tidx_v[pl.ds(c * nb, 16)]
            cv = cidx_v[pl.ds(c * nb, 16)]
            copies = []
            for l in range(nb):
                toff = pl.multiple_of((tv[l] >> 7) << 7, 128)
                coff = pl.multiple_of((cv[l] >> 7) << 7, 128)
                copies.append(pltpu.async_copy(
                    tableT_hbm.at[:, pl.ds(toff, 128)], tstage.at[l], sem_t))
                copies.append(pltpu.async_copy(
                    tableT_hbm.at[:, pl.ds(coff, 128)], cstage.at[l], sem_c))
            for cp in copies:
                cp.wait()
            pltpu.sync_copy(tstage, t_out.at[pl.ds(base + c * nb, nb)])
            pltpu.sync_copy(cstage, c_out.at[pl.ds(base + c * nb, nb)])

    return gather2


_gather2 = _build_gather()

_BM = 512


def _sel_body(t_ref, ts_ref, c_ref, cs_ref, to_ref, co_ref):
    lanes = lax.broadcasted_iota(jnp.int32, (1, 1, 128), 2)
    ts = ts_ref[...][:, :, None]
    cs = cs_ref[...][:, :, None]
    tm = (lanes == ts).astype(jnp.float32)
    cm = (lanes == cs).astype(jnp.float32)
    to_ref[...] = jnp.sum(t_ref[...] * tm, axis=2)
    co_ref[...] = jnp.sum(c_ref[...] * cm, axis=2)


_select = pl.pallas_call(
    _sel_body,
    grid=(B // _BM,),
    in_specs=[
        pl.BlockSpec((_BM, D, 128), lambda i: (i, 0, 0)),
        pl.BlockSpec((_BM, 1), lambda i: (i, 0)),
        pl.BlockSpec((_BM, D, 128), lambda i: (i, 0, 0)),
        pl.BlockSpec((_BM, 1), lambda i: (i, 0)),
    ],
    out_specs=[
        pl.BlockSpec((_BM, D), lambda i: (i, 0)),
        pl.BlockSpec((_BM, D), lambda i: (i, 0)),
    ],
    out_shape=[
        jax.ShapeDtypeStruct((B, D), jnp.float32),
        jax.ShapeDtypeStruct((B, D), jnp.float32),
    ],
)


def _mm_body(t_ref, c_ref, o_ref):
    o_ref[...] = lax.dot_general(
        t_ref[...], c_ref[...],
        dimension_numbers=(((1,), (1,)), ((), ())),
        preferred_element_type=jnp.float32,
    )


_matmul = pl.pallas_call(
    _mm_body,
    grid=(B // _BM,),
    in_specs=[
        pl.BlockSpec((_BM, D), lambda i: (i, 0)),
        pl.BlockSpec((B, D), lambda i: (0, 0)),
    ],
    out_specs=pl.BlockSpec((_BM, B), lambda i: (i, 0)),
    out_shape=jax.ShapeDtypeStruct((B, B), jnp.float32),
)


def kernel(target, context, table):
    target = target.astype(jnp.int32)
    context = context.astype(jnp.int32)
    t_tiles, c_tiles = _gather2(target, context, table.T)
    sel_t = (target & 127)[:, None]
    sel_c = (context & 127)[:, None]
    t_emb, c_emb = _select(t_tiles, sel_t, c_tiles, sel_c)
    return _matmul(t_emb, c_emb)


# speedup vs baseline: 4.1080x; 1.0266x over previous
import functools

import jax
import jax.numpy as jnp
from jax import lax
from jax.experimental import pallas as pl
from jax.experimental.pallas import tpu as pltpu
from jax.experimental.pallas import tpu_sc as plsc

B = 4096
D = 16
VOCAB = 1000000


def _build_gather():
    info = plsc.get_sparse_core_info()
    nc, ns = info.num_cores, info.num_subcores
    nw = nc * ns
    bpw = B // nw
    nb = 16
    mesh = plsc.VectorSubcoreMesh(core_axis_name="c", subcore_axis_name="s")

    @functools.partial(
        pl.kernel,
        mesh=mesh,
        out_type=[
            jax.ShapeDtypeStruct((B, D, 128), jnp.float32),
            jax.ShapeDtypeStruct((B, D, 128), jnp.float32),
        ],
        scratch_types=[
            pltpu.VMEM((bpw,), jnp.int32),
            pltpu.VMEM((bpw,), jnp.int32),
            pltpu.VMEM((nb, D, 128), jnp.float32),
            pltpu.VMEM((nb, D, 128), jnp.float32),
            pltpu.SemaphoreType.DMA,
            pltpu.SemaphoreType.DMA,
        ],
    )
    def gather2(tgt_hbm, ctx_hbm, tableT_hbm, t_out, c_out,
                tidx_v, cidx_v, tstage, cstage, sem_t, sem_c):
        wid = lax.axis_index("s") * nc + lax.axis_index("c")
        base = wid * bpw
        pltpu.sync_copy(tgt_hbm.at[pl.ds(base, bpw)], tidx_v)
        pltpu.sync_copy(ctx_hbm.at[pl.ds(base, bpw)], cidx_v)
        for c in range(bpw // nb):
            tv = tidx_v[pl.ds(c * nb, 16)]
            cv = cidx_v[pl.ds(c * nb, 16)]
            copies = []
            for l in range(nb):
                toff = pl.multiple_of((tv[l] >> 7) << 7, 128)
                coff = pl.multiple_of((cv[l] >> 7) << 7, 128)
                copies.append(pltpu.async_copy(
                    tableT_hbm.at[:, pl.ds(toff, 128)], tstage.at[l], sem_t))
                copies.append(pltpu.async_copy(
                    tableT_hbm.at[:, pl.ds(coff, 128)], cstage.at[l], sem_c))
            for cp in copies:
                cp.wait()
            pltpu.sync_copy(tstage, t_out.at[pl.ds(base + c * nb, nb)])
            pltpu.sync_copy(cstage, c_out.at[pl.ds(base + c * nb, nb)])

    return gather2


_gather2 = _build_gather()

_BM = 512
_NB = B // _BM


def _lane_select(tiles, sel):
    lanes = lax.broadcasted_iota(jnp.int32, (1, 1, 128), 2)
    m = (lanes == sel[:, :, None]).astype(jnp.float32)
    return jnp.sum(tiles * m, axis=2)


def _mm_body(tt_ref, ts_ref, ct_ref, cs_ref, o_ref, csel_ref):
    i = pl.program_id(0)

    @pl.when(i < _NB)
    def _select_c():
        csel_ref[pl.ds(i * _BM, _BM), :] = _lane_select(ct_ref[...],
                                                        cs_ref[...])

    @pl.when(i >= _NB)
    def _mm():
        t_sel = _lane_select(tt_ref[...], ts_ref[...])
        o_ref[...] = lax.dot_general(
            t_sel, csel_ref[...],
            dimension_numbers=(((1,), (1,)), ((), ())),
            preferred_element_type=jnp.float32,
        )


_matmul = pl.pallas_call(
    _mm_body,
    grid=(2 * _NB,),
    in_specs=[
        pl.BlockSpec((_BM, D, 128), lambda i: (jnp.maximum(i - _NB, 0), 0, 0)),
        pl.BlockSpec((_BM, 1), lambda i: (jnp.maximum(i - _NB, 0), 0)),
        pl.BlockSpec((_BM, D, 128), lambda i: (jnp.minimum(i, _NB - 1), 0, 0)),
        pl.BlockSpec((_BM, 1), lambda i: (jnp.minimum(i, _NB - 1), 0)),
    ],
    out_specs=pl.BlockSpec((_BM, B), lambda i: (jnp.maximum(i - _NB, 0), 0)),
    out_shape=jax.ShapeDtypeStruct((B, B), jnp.float32),
    scratch_shapes=[pltpu.VMEM((B, D), jnp.float32)],
)


def kernel(target, context, table):
    target = target.astype(jnp.int32)
    context = context.astype(jnp.int32)
    t_tiles, c_tiles = _gather2(target, context, table.T)
    sel_t = (target & 127)[:, None]
    sel_c = (context & 127)[:, None]
    return _matmul(t_tiles, sel_t, c_tiles, sel_c)


# speedup vs baseline: 4.1277x; 1.0048x over previous
import functools

import jax
import jax.numpy as jnp
from jax import lax
from jax.experimental import pallas as pl
from jax.experimental.pallas import tpu as pltpu
from jax.experimental.pallas import tpu_sc as plsc

B = 4096
D = 16
VOCAB = 1000000


def _build_gather():
    info = plsc.get_sparse_core_info()
    nc, ns = info.num_cores, info.num_subcores
    nw = nc * ns
    bpw = B // nw
    nb = 16
    mesh = plsc.VectorSubcoreMesh(core_axis_name="c", subcore_axis_name="s")

    @functools.partial(
        pl.kernel,
        mesh=mesh,
        out_type=jax.ShapeDtypeStruct((B, D, 128), jnp.float32),
        scratch_types=[
            pltpu.VMEM((bpw,), jnp.int32),
            pltpu.VMEM((nb, D, 128), jnp.float32),
            pltpu.SemaphoreType.DMA,
        ],
    )
    def gather1(idx_hbm, tableT_hbm, tiles_out, idx_v, stage, sem):
        wid = lax.axis_index("s") * nc + lax.axis_index("c")
        base = wid * bpw
        pltpu.sync_copy(idx_hbm.at[pl.ds(base, bpw)], idx_v)
        for c in range(bpw // nb):
            v = idx_v[pl.ds(c * nb, 16)]
            copies = []
            for l in range(nb):
                off = pl.multiple_of((v[l] >> 7) << 7, 128)
                copies.append(pltpu.async_copy(
                    tableT_hbm.at[:, pl.ds(off, 128)], stage.at[l], sem))
            for cp in copies:
                cp.wait()
            pltpu.sync_copy(stage, tiles_out.at[pl.ds(base + c * nb, nb)])

    return gather1


_gather1 = _build_gather()

_BM = 512
_NB = B // _BM


def _lane_select(tiles, sel):
    lanes = lax.broadcasted_iota(jnp.int32, (1, 1, 128), 2)
    m = (lanes == sel[:, :, None]).astype(jnp.float32)
    return jnp.sum(tiles * m, axis=2)


def _sel_body(ct_ref, cs_ref, co_ref):
    co_ref[...] = _lane_select(ct_ref[...], cs_ref[...])


_select_c = pl.pallas_call(
    _sel_body,
    grid=(_NB,),
    in_specs=[
        pl.BlockSpec((_BM, D, 128), lambda i: (i, 0, 0)),
        pl.BlockSpec((_BM, 1), lambda i: (i, 0)),
    ],
    out_specs=pl.BlockSpec((_BM, D), lambda i: (i, 0)),
    out_shape=jax.ShapeDtypeStruct((B, D), jnp.float32),
)


def _mm_body(tt_ref, ts_ref, c_ref, o_ref):
    t_sel = _lane_select(tt_ref[...], ts_ref[...])
    o_ref[...] = lax.dot_general(
        t_sel, c_ref[...],
        dimension_numbers=(((1,), (1,)), ((), ())),
        preferred_element_type=jnp.float32,
    )


_matmul = pl.pallas_call(
    _mm_body,
    grid=(_NB,),
    in_specs=[
        pl.BlockSpec((_BM, D, 128), lambda i: (i, 0, 0)),
        pl.BlockSpec((_BM, 1), lambda i: (i, 0)),
        pl.BlockSpec((B, D), lambda i: (0, 0)),
    ],
    out_specs=pl.BlockSpec((_BM, B), lambda i: (i, 0)),
    out_shape=jax.ShapeDtypeStruct((B, B), jnp.float32),
)


def kernel(target, context, table):
    target = target.astype(jnp.int32)
    context = context.astype(jnp.int32)
    tableT = table.T
    c_tiles = _gather1(context, tableT)
    t_tiles = _gather1(target, tableT)
    c_emb = _select_c(c_tiles, (context & 127)[:, None])
    return _matmul(t_tiles, (target & 127)[:, None], c_emb)
